# baseline (speedup 1.0000x reference)
import functools

import jax
import jax.numpy as jnp
from jax.experimental import pallas as pl
from jax.experimental.pallas import tpu as pltpu

S = 2048
D = 2048
H, DH = 16, 128
HI, DI = 4, 64
TOPK = 512
NEG = -1e30

QB = 256
RB = 256
KB = 512
PCOLS = 3 * H * DH + HI * DI + 128


def _bf(a):
    return a.astype(jnp.bfloat16)


def _proj_kernel(x_ref, wq_ref, wk_ref, wv_ref, wiq_ref, wik_ref, o_ref):
    xb = _bf(x_ref[...])
    scale = 1.4426950408889634 / jnp.sqrt(jnp.float32(DH))
    for lo, w_ref, sc in ((0, wq_ref, scale), (2048, wk_ref, None),
                          (4096, wv_ref, None), (6144, wiq_ref, None),
                          (6400, wik_ref, None)):
        r = jnp.dot(xb, w_ref[...], preferred_element_type=jnp.float32)
        if sc is not None:
            r = r * sc
        o_ref[:, lo:lo + w_ref.shape[1]] = _bf(r)
    o_ref[:, 6464:] = jnp.zeros((RB, PCOLS - 6464), jnp.bfloat16)


def _attn_kernel(b0, kw, q_ref, k_ref, v_ref, iq_ref, ik_ref, w2_ref,
                 wo_ref, o_ref):
    i = pl.program_id(0)

    ik = ik_ref[:, :DI]
    iq2 = iq_ref[...]
    s2 = jax.lax.dot_general(iq2, ik, (((1,), (1,)), ((), ())),
                             preferred_element_type=jnp.float32)
    r2 = _bf(jnp.maximum(s2, 0.0))
    isc = jnp.dot(w2_ref[...], r2, preferred_element_type=jnp.float32)

    rows = (b0 + i) * QB + jax.lax.broadcasted_iota(jnp.int32, (QB, kw), 0)
    cols = jax.lax.broadcasted_iota(jnp.int32, (QB, kw), 1)
    causal = cols <= rows
    isc = jnp.where(causal, isc, NEG)

    bits = jax.lax.bitcast_convert_type(isc, jnp.int32)
    key = jnp.where(bits < 0, bits ^ jnp.int32(0x7FFFFFFF), bits)
    cand = jnp.full((QB, 1), jnp.int32(-2**31))
    for b in range(31, -1, -1):
        mask_b = jnp.int32(-2**31) if b == 31 else jnp.int32(1 << b)
        trial = cand ^ mask_b
        cnt = jnp.sum((key >= trial).astype(jnp.int32), axis=1,
                      keepdims=True)
        cand = jnp.where(cnt >= TOPK, trial, cand)
    sel = (key >= cand) & causal
    madd = jnp.where(sel, -43.3, NEG).astype(jnp.float32)

    outs = []
    for h in range(H):
        q_h = q_ref[:, h * DH:(h + 1) * DH]
        k_h = k_ref[:, h * DH:(h + 1) * DH]
        v_h = v_ref[:, h * DH:(h + 1) * DH]
        logits = jax.lax.dot_general(q_h, k_h, (((1,), (1,)), ((), ())),
                                     preferred_element_type=jnp.float32)
        p = jnp.exp2(logits + madd)
        denom = jnp.sum(p, axis=1, keepdims=True)
        o_h = jnp.dot(_bf(p), v_h,
                      preferred_element_type=jnp.float32) / denom
        outs.append(_bf(o_h))
    ob = jnp.concatenate(outs, axis=1)
    o_ref[...] = jnp.dot(ob, wo_ref[...], preferred_element_type=jnp.float32)


def kernel(x, wq, wk, wv, wo, wiq, wik, w_ih):
    qkv = pl.pallas_call(
        _proj_kernel,
        grid=(S // RB,),
        in_specs=[
            pl.BlockSpec((RB, D), lambda j: (j, 0)),
            pl.BlockSpec((D, H * DH), lambda j: (0, 0)),
            pl.BlockSpec((D, H * DH), lambda j: (0, 0)),
            pl.BlockSpec((D, H * DH), lambda j: (0, 0)),
            pl.BlockSpec((D, HI * DI), lambda j: (0, 0)),
            pl.BlockSpec((D, DI), lambda j: (0, 0)),
        ],
        out_specs=pl.BlockSpec((RB, PCOLS), lambda j: (j, 0)),
        out_shape=jax.ShapeDtypeStruct((S, PCOLS), jnp.bfloat16),
    )(x[0], _bf(wq), _bf(wk), _bf(wv), _bf(wiq), _bf(wik))

    w2 = _bf(jnp.kron(jnp.eye(QB, dtype=jnp.float32), w_ih.reshape(1, HI)))
    wob = _bf(wo)
    iq2_all = qkv[:, 6144:6400].reshape(S * HI, DI)
    pieces = []
    for b0, nb in ((0, 2), (2, 2), (4, 2), (6, 2)):
        kw = (b0 + nb) * QB
        piece = pl.pallas_call(
            functools.partial(_attn_kernel, b0, kw),
            grid=(nb,),
            in_specs=[
                pl.BlockSpec((QB, H * DH), lambda i, b0=b0: (b0 + i, 0)),
                pl.BlockSpec((kw, H * DH), lambda i: (0, 1)),
                pl.BlockSpec((kw, H * DH), lambda i: (0, 2)),
                pl.BlockSpec((QB * HI, DI), lambda i, b0=b0: (b0 + i, 0)),
                pl.BlockSpec((kw, 128), lambda i: (0, 50)),
                pl.BlockSpec((QB, QB * HI), lambda i: (0, 0)),
                pl.BlockSpec((D, D), lambda i: (0, 0)),
            ],
            out_specs=pl.BlockSpec((QB, D), lambda i: (i, 0)),
            out_shape=jax.ShapeDtypeStruct((nb * QB, D), jnp.float32),
        )(qkv, qkv, qkv, iq2_all, qkv, w2, wob)
        pieces.append(piece)
    out = jnp.concatenate(pieces, axis=0)
    return out.reshape(1, S, D)


# speedup vs baseline: 1.0439x; 1.0439x over previous
import functools

import jax
import jax.numpy as jnp
from jax.experimental import pallas as pl
from jax.experimental.pallas import tpu as pltpu

S = 2048
D = 2048
H, DH = 16, 128
HI, DI = 4, 64
TOPK = 512
NEG = -1e30

QB = 256
RB = 256
KB = 512
PCOLS = 3 * H * DH + HI * DI + 128


def _bf(a):
    return a.astype(jnp.bfloat16)


def _proj_kernel(x_ref, wq_ref, wk_ref, wv_ref, wiq_ref, wik_ref, o_ref):
    xb = _bf(x_ref[...])
    scale = 1.4426950408889634 / jnp.sqrt(jnp.float32(DH))
    for lo, w_ref, sc in ((0, wq_ref, scale), (2048, wk_ref, None),
                          (4096, wv_ref, None), (6144, wiq_ref, None),
                          (6400, wik_ref, None)):
        r = jnp.dot(xb, w_ref[...], preferred_element_type=jnp.float32)
        if sc is not None:
            r = r * sc
        o_ref[:, lo:lo + w_ref.shape[1]] = _bf(r)
    o_ref[:, 6464:] = jnp.zeros((RB, PCOLS - 6464), jnp.bfloat16)


def _attn_kernel(b0, kw, q_ref, k_ref, v_ref, iq_ref, ik_ref, wih_ref,
                 wo_ref, o_ref):
    i = pl.program_id(0)

    ik = ik_ref[:, :DI]
    isc = None
    for h in range(HI):
        iq_h = iq_ref[:, h * DI:(h + 1) * DI]
        s = jax.lax.dot_general(iq_h, ik, (((1,), (1,)), ((), ())),
                                preferred_element_type=jnp.float32)
        r = (_bf(jnp.maximum(s, 0.0)).astype(jnp.float32)
             * _bf(wih_ref[0, h]).astype(jnp.float32))
        isc = r if isc is None else isc + r

    rows = (b0 + i) * QB + jax.lax.broadcasted_iota(jnp.int32, (QB, kw), 0)
    cols = jax.lax.broadcasted_iota(jnp.int32, (QB, kw), 1)
    causal = cols <= rows
    isc = jnp.where(causal, isc, NEG)

    bits = jax.lax.bitcast_convert_type(isc, jnp.int32)
    key = jnp.where(bits < 0, bits ^ jnp.int32(0x7FFFFFFF), bits)
    cand = jnp.full((QB, 1), jnp.int32(-2**31))
    for b in range(31, -1, -1):
        mask_b = jnp.int32(-2**31) if b == 31 else jnp.int32(1 << b)
        trial = cand ^ mask_b
        cnt = jnp.sum((key >= trial).astype(jnp.int32), axis=1,
                      keepdims=True)
        cand = jnp.where(cnt >= TOPK, trial, cand)
    sel = (key >= cand) & causal
    madd = jnp.where(sel, -43.3, NEG).astype(jnp.float32)

    outs = []
    for h in range(H):
        q_h = q_ref[:, h * DH:(h + 1) * DH]
        k_h = k_ref[:, h * DH:(h + 1) * DH]
        v_h = v_ref[:, h * DH:(h + 1) * DH]
        logits = jax.lax.dot_general(q_h, k_h, (((1,), (1,)), ((), ())),
                                     preferred_element_type=jnp.float32)
        p = jnp.exp2(logits + madd)
        denom = jnp.sum(p, axis=1, keepdims=True)
        o_h = jnp.dot(_bf(p), v_h,
                      preferred_element_type=jnp.float32) / denom
        outs.append(_bf(o_h))
    ob = jnp.concatenate(outs, axis=1)
    o_ref[...] = jnp.dot(ob, wo_ref[...], preferred_element_type=jnp.float32)


def kernel(x, wq, wk, wv, wo, wiq, wik, w_ih):
    qkv = pl.pallas_call(
        _proj_kernel,
        grid=(S // RB,),
        in_specs=[
            pl.BlockSpec((RB, D), lambda j: (j, 0)),
            pl.BlockSpec((D, H * DH), lambda j: (0, 0)),
            pl.BlockSpec((D, H * DH), lambda j: (0, 0)),
            pl.BlockSpec((D, H * DH), lambda j: (0, 0)),
            pl.BlockSpec((D, HI * DI), lambda j: (0, 0)),
            pl.BlockSpec((D, DI), lambda j: (0, 0)),
        ],
        out_specs=pl.BlockSpec((RB, PCOLS), lambda j: (j, 0)),
        out_shape=jax.ShapeDtypeStruct((S, PCOLS), jnp.bfloat16),
    )(x[0], _bf(wq), _bf(wk), _bf(wv), _bf(wiq), _bf(wik))

    wih2 = jnp.pad(w_ih.reshape(1, HI), ((0, 0), (0, 128 - HI)))
    wob = _bf(wo)
    pieces = []
    for b0, nb in ((0, 4), (4, 4)):
        kw = (b0 + nb) * QB
        piece = pl.pallas_call(
            functools.partial(_attn_kernel, b0, kw),
            grid=(nb,),
            in_specs=[
                pl.BlockSpec((QB, H * DH), lambda i, b0=b0: (b0 + i, 0)),
                pl.BlockSpec((kw, H * DH), lambda i: (0, 1)),
                pl.BlockSpec((kw, H * DH), lambda i: (0, 2)),
                pl.BlockSpec((QB, HI * DI), lambda i, b0=b0: (b0 + i, 24)),
                pl.BlockSpec((kw, 128), lambda i: (0, 50)),
                pl.BlockSpec((1, 128), lambda i: (0, 0)),
                pl.BlockSpec((D, D), lambda i: (0, 0)),
            ],
            out_specs=pl.BlockSpec((QB, D), lambda i: (i, 0)),
            out_shape=jax.ShapeDtypeStruct((nb * QB, D), jnp.float32),
        )(qkv, qkv, qkv, qkv, qkv, wih2, wob)
        pieces.append(piece)
    out = jnp.concatenate(pieces, axis=0)
    return out.reshape(1, S, D)
